# baseline (speedup 1.0000x reference)
import functools

import jax
import jax.numpy as jnp
from jax import lax
from jax.experimental import pallas as pl
from jax.experimental.pallas import tpu as pltpu
from jax.experimental.pallas import tpu_sc as plsc

NC, NS = 2, 16
NW = NC * NS
C = 128
D = 32


def _make_kernel(N, V, K):
    nblk = N // K
    assert nblk * K == N
    rounds = nblk // NW
    assert rounds * NW == nblk
    mesh = plsc.VectorSubcoreMesh(core_axis_name="c", subcore_axis_name="s")

    @functools.partial(
        pl.kernel,
        out_type=jax.ShapeDtypeStruct((N, D), jnp.float32),
        mesh=mesh,
        scratch_types=[
            pltpu.VMEM((K,), jnp.int32),
            pltpu.VMEM((K, D), jnp.float32),
            pltpu.SemaphoreType.DMA,
        ],
        compiler_params=pltpu.CompilerParams(use_tc_tiling_on_sc=False),
    )
    def k(table_hbm, idx_hbm, out_hbm, idx_v, rows, sem):
        wid = lax.axis_index("s") * NC + lax.axis_index("c")

        def step(r, carry):
            blk = r * NW + wid
            pltpu.sync_copy(idx_hbm.at[pl.ds(blk * K, K)], idx_v)
            copies = [
                pltpu.async_copy(
                    table_hbm.at[idx_v.at[pl.ds(c * C, C)]],
                    rows.at[pl.ds(c * C, C)],
                    sem,
                )
                for c in range(K // C)
            ]
            for cp in copies:
                cp.wait()
            pltpu.sync_copy(rows, out_hbm.at[pl.ds(blk * K, K)])
            return carry

        lax.fori_loop(0, rounds, step, 0)

    return k


def kernel(inputs, weight):
    B, S = inputs.shape
    V, D_ = weight.shape
    flat = inputs.reshape(-1).astype(jnp.int32)
    out = _make_kernel(B * S, V, 1024)(weight, flat)
    return out.reshape(B, S, D_)


# speedup vs baseline: 1.0119x; 1.0119x over previous
import functools

import jax
import jax.numpy as jnp
from jax import lax
from jax.experimental import pallas as pl
from jax.experimental.pallas import tpu as pltpu
from jax.experimental.pallas import tpu_sc as plsc

NC, NS = 2, 16
NW = NC * NS
C = 128
D = 32
NBUF = 4


def _make_kernel(N, V, K):
    nblk = N // K
    assert nblk * K == N
    rounds = nblk // NW
    assert rounds * NW == nblk
    assert rounds % NBUF == 0 and rounds // NBUF >= 2
    groups = rounds // NBUF
    CH = K // C
    assert CH * C == K
    mesh = plsc.VectorSubcoreMesh(core_axis_name="c", subcore_axis_name="s")

    @functools.partial(
        pl.kernel,
        out_type=jax.ShapeDtypeStruct((N, D), jnp.float32),
        mesh=mesh,
        scratch_types=[
            pltpu.VMEM((NBUF, K), jnp.int32),
            pltpu.VMEM((K, D), jnp.float32),
            pltpu.VMEM((K, D), jnp.float32),
            pltpu.VMEM((K, D), jnp.float32),
            pltpu.VMEM((K, D), jnp.float32),
        ]
        + [pltpu.SemaphoreType.DMA] * (3 * NBUF),
        compiler_params=pltpu.CompilerParams(use_tc_tiling_on_sc=False),
    )
    def k(table_hbm, idx_hbm, out_hbm, idx_v, b0, b1, b2, b3, *sems):
        rows = [b0, b1, b2, b3]
        gs = sems[0:NBUF]
        os_ = sems[NBUF:2 * NBUF]
        is_ = sems[2 * NBUF:3 * NBUF]
        wid = lax.axis_index("s") * NC + lax.axis_index("c")

        def off(r):
            return (r * NW + wid) * K

        def fire_idx(r, j):
            pltpu.async_copy(idx_hbm.at[pl.ds(off(r), K)], idx_v.at[j], is_[j])

        def wait_idx(j):
            pltpu.make_async_copy(
                idx_hbm.at[pl.ds(0, K)], idx_v.at[j], is_[j]
            ).wait()

        def fire_g(j):
            for c in range(CH):
                pltpu.async_copy(
                    table_hbm.at[idx_v.at[j].at[pl.ds(c * C, C)]],
                    rows[j].at[pl.ds(c * C, C)],
                    gs[j],
                )

        def wait_g(j):
            pltpu.make_async_copy(
                table_hbm.at[pl.ds(0, K)], rows[j], gs[j]
            ).wait()

        def fire_s(r, j):
            pltpu.async_copy(rows[j], out_hbm.at[pl.ds(off(r), K)], os_[j])

        def wait_s(j):
            pltpu.make_async_copy(
                rows[j], out_hbm.at[pl.ds(0, K)], os_[j]
            ).wait()

        fire_idx(0, 0)
        for j in range(NBUF):
            wait_idx(j)
            fire_g(j)
            fire_idx(j + 1, (j + 1) % NBUF)
            if j > 0:
                wait_g(j - 1)
                fire_s(j - 1, j - 1)

        def group(g, carry):
            for j in range(NBUF):
                r = g * NBUF + j
                wait_s(j)
                wait_idx(j)
                fire_g(j)
                fire_idx(r + 1, (j + 1) % NBUF)
                jp = (j - 1) % NBUF
                wait_g(jp)
                fire_s(r - 1, jp)
            return carry

        lax.fori_loop(1, groups - 1, group, 0)

        gT = groups - 1
        for j in range(NBUF):
            r = gT * NBUF + j
            wait_s(j)
            wait_idx(j)
            fire_g(j)
            if j + 1 < NBUF:
                fire_idx(r + 1, (j + 1) % NBUF)
            jp = (j - 1) % NBUF
            wait_g(jp)
            fire_s(r - 1, jp)

        wait_g(NBUF - 1)
        fire_s(rounds - 1, NBUF - 1)
        for j in range(NBUF):
            wait_s(j)

    return k


def kernel(inputs, weight):
    B, S = inputs.shape
    V, D_ = weight.shape
    flat = inputs.reshape(-1).astype(jnp.int32)
    out = _make_kernel(B * S, V, 256)(weight, flat)
    return out.reshape(B, S, D_)


# speedup vs baseline: 1.7603x; 1.7396x over previous
import functools

import jax
import jax.numpy as jnp
from jax import lax
from jax.experimental import pallas as pl
from jax.experimental.pallas import tpu as pltpu
from jax.experimental.pallas import tpu_sc as plsc

NC, NS = 2, 16
NW = NC * NS
C = 128
D = 32
NBUF = 4


def _make_kernel(N, V, K):
    nblk = N // K
    assert nblk * K == N
    rounds = nblk // NW
    assert rounds * NW == nblk
    assert rounds % NBUF == 0 and rounds // NBUF >= 2
    groups = rounds // NBUF
    CH = K // C
    assert CH * C == K
    mesh = plsc.VectorSubcoreMesh(core_axis_name="c", subcore_axis_name="s")

    @functools.partial(
        pl.kernel,
        out_type=jax.ShapeDtypeStruct((N, D), jnp.float32),
        mesh=mesh,
        scratch_types=[
            pltpu.VMEM((NBUF, K), jnp.int32),
            pltpu.VMEM((K, D), jnp.float32),
            pltpu.VMEM((K, D), jnp.float32),
            pltpu.VMEM((K, D), jnp.float32),
            pltpu.VMEM((K, D), jnp.float32),
        ]
        + [pltpu.SemaphoreType.DMA] * (3 * NBUF),
        compiler_params=pltpu.CompilerParams(use_tc_tiling_on_sc=False),
    )
    def k(table_hbm, idx_hbm, out_hbm, idx_v, b0, b1, b2, b3, *sems):
        rows = [b0, b1, b2, b3]
        gs = sems[0:NBUF]
        os_ = sems[NBUF:2 * NBUF]
        is_ = sems[2 * NBUF:3 * NBUF]
        wid = lax.axis_index("s") * NC + lax.axis_index("c")

        def off(r):
            return (r * NW + wid) * K

        def fire_idx(r, j):
            pltpu.async_copy(idx_hbm.at[pl.ds(off(r), K)], idx_v.at[j], is_[j])

        def wait_idx(j):
            pltpu.make_async_copy(
                idx_hbm.at[pl.ds(0, K)], idx_v.at[j], is_[j]
            ).wait()

        def fire_g(j):
            for c in range(CH):
                pltpu.async_copy(
                    table_hbm.at[idx_v.at[j].at[pl.ds(c * C, C)]],
                    rows[j].at[pl.ds(c * C, C)],
                    gs[j],
                )

        def wait_g(j):
            pltpu.make_async_copy(
                table_hbm.at[pl.ds(0, K)], rows[j], gs[j]
            ).wait()

        def fire_s(r, j):
            pltpu.async_copy(rows[j], out_hbm.at[pl.ds(off(r), K)], os_[j])

        def wait_s(j):
            pltpu.make_async_copy(
                rows[j], out_hbm.at[pl.ds(0, K)], os_[j]
            ).wait()

        fire_idx(0, 0)
        for j in range(NBUF):
            wait_idx(j)
            fire_g(j)
            fire_idx(j + 1, (j + 1) % NBUF)
            if j > 0:
                wait_g(j - 1)
                fire_s(j - 1, j - 1)

        def group(g, carry):
            for j in range(NBUF):
                r = g * NBUF + j
                wait_s(j)
                wait_idx(j)
                fire_g(j)
                fire_idx(r + 1, (j + 1) % NBUF)
                jp = (j - 1) % NBUF
                wait_g(jp)
                fire_s(r - 1, jp)
            return carry

        lax.fori_loop(1, groups - 1, group, 0)

        gT = groups - 1
        for j in range(NBUF):
            r = gT * NBUF + j
            wait_s(j)
            wait_idx(j)
            fire_g(j)
            if j + 1 < NBUF:
                fire_idx(r + 1, (j + 1) % NBUF)
            jp = (j - 1) % NBUF
            wait_g(jp)
            fire_s(r - 1, jp)

        wait_g(NBUF - 1)
        fire_s(rounds - 1, NBUF - 1)
        for j in range(NBUF):
            wait_s(j)

    return k


def kernel(inputs, weight):
    B, S = inputs.shape
    V, D_ = weight.shape
    flat = jnp.transpose(inputs).reshape(-1).astype(jnp.int32)
    out = _make_kernel(B * S, V, 256)(weight, flat)
    return jnp.transpose(out.reshape(S, B, D_), (1, 0, 2))
